# baseline (speedup 1.0000x reference)
import functools

import jax
import jax.numpy as jnp
from jax.experimental import pallas as pl
from jax.experimental.pallas import tpu as pltpu

NUM_SHAPE_CODES = 1024
NUM_COLOR_CODES = 16
EMBEDDING_DIM = 256
HALF_DIM = 128
COMMITMENT_COST = 0.25


def _vq_kernel(xsb_ref, xc_ref, wsa_ref, ws_ref, wc_ref,
               out_ref, loss_ref, ps_ref, pc_ref,
               cs_acc, cc_acc, ls_acc, *, num_blocks, n_total, sub):
    b = pl.program_id(0)
    wsa = wsa_ref[...]
    ws = ws_ref[...]
    wc = wc_ref[...]

    @pl.when(b == 0)
    def _init():
        cs_acc[...] = jnp.zeros_like(cs_acc)
        cc_acc[...] = jnp.zeros_like(cc_acc)
        ls_acc[...] = jnp.zeros_like(ls_acc)

    cs_part = cs_acc[...]
    cc_part = cc_acc[...]
    ls_part = ls_acc[...]
    for t in range(sub):
        xs_aug = xsb_ref[t]
        xc = xc_ref[t]
        p = xc.shape[1]

        ds = jax.lax.dot_general(wsa, xs_aug, (((1,), (0,)), ((), ())),
                                 preferred_element_type=jnp.float32)
        ms = jnp.min(ds, axis=0, keepdims=True)
        es = jnp.where(ds == ms, 1.0, 0.0)

        w2c = jnp.sum(wc * wc, axis=1, keepdims=True)
        x2c = jnp.sum(xc * xc, axis=0, keepdims=True)
        sc = jax.lax.dot_general(wc, xc, (((1,), (0,)), ((), ())),
                                 preferred_element_type=jnp.float32)
        dc = (x2c + w2c) - 2.0 * sc
        mc = jnp.min(dc, axis=0, keepdims=True)
        iota_c = jax.lax.broadcasted_iota(jnp.int32, (NUM_COLOR_CODES, p), 0)
        idx_c = jnp.min(jnp.where(dc == mc, iota_c, NUM_COLOR_CODES),
                        axis=0, keepdims=True)
        ec = jnp.where(iota_c == idx_c, 1.0, 0.0)

        qs = jax.lax.dot_general(ws, es, (((0,), (0,)), ((), ())),
                                 preferred_element_type=jnp.float32)
        qc = jax.lax.dot_general(wc, ec, (((0,), (0,)), ((), ())),
                                 preferred_element_type=jnp.float32)
        out_ref[t, :HALF_DIM, :] = qs
        out_ref[t, HALF_DIM:, :] = qc

        xs32 = xs_aug[:HALF_DIM, :].astype(jnp.float32)
        x2s_row = jnp.sum(xs32 * xs32, axis=0, keepdims=True)
        ls_part = ls_part + (ms + x2s_row + mc)

        es_part = es[:, 0:HALF_DIM]
        ec_part = ec[:, 0:HALF_DIM]
        for k in range(1, p // HALF_DIM):
            es_part = es_part + es[:, k * HALF_DIM:(k + 1) * HALF_DIM]
            ec_part = ec_part + ec[:, k * HALF_DIM:(k + 1) * HALF_DIM]
        cs_part = cs_part + es_part
        cc_part = cc_part + ec_part
    cs_acc[...] = cs_part
    cc_acc[...] = cc_part
    ls_acc[...] = ls_part

    @pl.when(b == num_blocks - 1)
    def _finalize():
        probs_s = jnp.sum(cs_acc[...], axis=1, keepdims=True) * (1.0 / n_total)
        probs_c = jnp.sum(cc_acc[...], axis=1, keepdims=True) * (1.0 / n_total)
        ps_ref[...] = jnp.exp(-jnp.sum(probs_s * jnp.log(probs_s + 1e-10))).reshape(1, 1)
        pc_ref[...] = jnp.exp(-jnp.sum(probs_c * jnp.log(probs_c + 1e-10))).reshape(1, 1)
        scale = (1.0 + COMMITMENT_COST) / (n_total * EMBEDDING_DIM)
        loss_ref[...] = (jnp.sum(ls_acc[...]) * scale).reshape(1, 1)


def kernel(inputs, W_shape, W_color):
    batch, emb, h, w = inputs.shape
    hw = h * w
    n_total = batch * hw
    x3 = inputs.reshape(batch, emb, hw)
    xsb = jnp.concatenate(
        [x3[:, :HALF_DIM, :].astype(jnp.bfloat16),
         jnp.ones((batch, 1, hw), jnp.bfloat16)], axis=1)
    xc = x3[:, HALF_DIM:, :]
    ws_aug = jnp.concatenate(
        [W_shape * -2.0, jnp.sum(W_shape * W_shape, axis=1, keepdims=True)],
        axis=1).astype(jnp.bfloat16)

    sub = 4
    grid = (batch // sub,)
    kfn = functools.partial(_vq_kernel, num_blocks=batch // sub,
                            n_total=n_total, sub=sub)
    out, loss, ps, pc = pl.pallas_call(
        kfn,
        grid=grid,
        in_specs=[
            pl.BlockSpec((sub, HALF_DIM + 1, hw), lambda b: (b, 0, 0)),
            pl.BlockSpec((sub, HALF_DIM, hw), lambda b: (b, 0, 0)),
            pl.BlockSpec((NUM_SHAPE_CODES, HALF_DIM + 1), lambda b: (0, 0)),
            pl.BlockSpec((NUM_SHAPE_CODES, HALF_DIM), lambda b: (0, 0)),
            pl.BlockSpec((NUM_COLOR_CODES, HALF_DIM), lambda b: (0, 0)),
        ],
        out_specs=[
            pl.BlockSpec((sub, emb, hw), lambda b: (b, 0, 0)),
            pl.BlockSpec((1, 1), lambda b: (0, 0)),
            pl.BlockSpec((1, 1), lambda b: (0, 0)),
            pl.BlockSpec((1, 1), lambda b: (0, 0)),
        ],
        out_shape=[
            jax.ShapeDtypeStruct((batch, emb, hw), jnp.float32),
            jax.ShapeDtypeStruct((1, 1), jnp.float32),
            jax.ShapeDtypeStruct((1, 1), jnp.float32),
            jax.ShapeDtypeStruct((1, 1), jnp.float32),
        ],
        scratch_shapes=[
            pltpu.VMEM((NUM_SHAPE_CODES, HALF_DIM), jnp.float32),
            pltpu.VMEM((NUM_COLOR_CODES, HALF_DIM), jnp.float32),
            pltpu.VMEM((1, hw), jnp.float32),
        ],
        compiler_params=pltpu.CompilerParams(
            dimension_semantics=("arbitrary",),
        ),
    )(xsb, xc, ws_aug, W_shape, W_color)

    quantized = out.reshape(batch, emb, h, w)
    return (quantized, loss[0, 0], ps[0, 0], pc[0, 0])


# speedup vs baseline: 1.3226x; 1.3226x over previous
import functools

import jax
import jax.numpy as jnp
from jax.experimental import pallas as pl
from jax.experimental.pallas import tpu as pltpu

NUM_SHAPE_CODES = 1024
NUM_COLOR_CODES = 16
EMBEDDING_DIM = 256
HALF_DIM = 128
COMMITMENT_COST = 0.25


def _vq_kernel(x_ref, wsa_ref, ws_ref, wc_ref,
               out_ref, loss_ref, ps_ref, pc_ref,
               cs_acc, cc_acc, ls_acc, *, num_blocks, n_total, sub):
    b = pl.program_id(0)
    wsa = wsa_ref[...]
    ws = ws_ref[...]
    wc = wc_ref[...]

    @pl.when(b == 0)
    def _init():
        cs_acc[...] = jnp.zeros_like(cs_acc)
        cc_acc[...] = jnp.zeros_like(cc_acc)
        ls_acc[...] = jnp.zeros_like(ls_acc)

    cs_part = cs_acc[...]
    cc_part = cc_acc[...]
    ls_part = ls_acc[...]
    for t in range(sub):
        x = x_ref[t]
        xs = x[:HALF_DIM, :]
        xc = x[HALF_DIM:, :]
        p = x.shape[1]

        xs_aug = jnp.concatenate(
            [xs.astype(jnp.bfloat16), jnp.ones((1, p), jnp.bfloat16)], axis=0)
        ds = jax.lax.dot_general(wsa, xs_aug, (((1,), (0,)), ((), ())),
                                 preferred_element_type=jnp.float32)
        ms = jnp.min(ds, axis=0, keepdims=True)
        es = jnp.where(ds == ms, 1.0, 0.0)

        w2c = jnp.sum(wc * wc, axis=1, keepdims=True)
        x2c = jnp.sum(xc * xc, axis=0, keepdims=True)
        sc = jax.lax.dot_general(wc, xc, (((1,), (0,)), ((), ())),
                                 preferred_element_type=jnp.float32)
        dc = (x2c + w2c) - 2.0 * sc
        mc = jnp.min(dc, axis=0, keepdims=True)
        iota_c = jax.lax.broadcasted_iota(jnp.int32, (NUM_COLOR_CODES, p), 0)
        idx_c = jnp.min(jnp.where(dc == mc, iota_c, NUM_COLOR_CODES),
                        axis=0, keepdims=True)
        ec = jnp.where(iota_c == idx_c, 1.0, 0.0)

        qs = jax.lax.dot_general(ws, es, (((0,), (0,)), ((), ())),
                                 preferred_element_type=jnp.float32)
        qc = jax.lax.dot_general(wc, ec, (((0,), (0,)), ((), ())),
                                 preferred_element_type=jnp.float32)
        out_ref[t, :HALF_DIM, :] = qs
        out_ref[t, HALF_DIM:, :] = qc

        x2s_row = jnp.sum(xs * xs, axis=0, keepdims=True)
        ls_part = ls_part + (ms + x2s_row + mc)

        es_part = es[:, 0:HALF_DIM]
        ec_part = ec[:, 0:HALF_DIM]
        for k in range(1, p // HALF_DIM):
            es_part = es_part + es[:, k * HALF_DIM:(k + 1) * HALF_DIM]
            ec_part = ec_part + ec[:, k * HALF_DIM:(k + 1) * HALF_DIM]
        cs_part = cs_part + es_part
        cc_part = cc_part + ec_part
    cs_acc[...] = cs_part
    cc_acc[...] = cc_part
    ls_acc[...] = ls_part

    @pl.when(b == num_blocks - 1)
    def _finalize():
        probs_s = jnp.sum(cs_acc[...], axis=1, keepdims=True) * (1.0 / n_total)
        probs_c = jnp.sum(cc_acc[...], axis=1, keepdims=True) * (1.0 / n_total)
        ps_ref[...] = jnp.exp(-jnp.sum(probs_s * jnp.log(probs_s + 1e-10))).reshape(1, 1)
        pc_ref[...] = jnp.exp(-jnp.sum(probs_c * jnp.log(probs_c + 1e-10))).reshape(1, 1)
        scale = (1.0 + COMMITMENT_COST) / (n_total * EMBEDDING_DIM)
        loss_ref[...] = (jnp.sum(ls_acc[...]) * scale).reshape(1, 1)


def kernel(inputs, W_shape, W_color):
    batch, emb, h, w = inputs.shape
    hw = h * w
    n_total = batch * hw
    x3 = inputs.reshape(batch, emb, hw)
    ws_aug = jnp.concatenate(
        [W_shape * -2.0, jnp.sum(W_shape * W_shape, axis=1, keepdims=True)],
        axis=1).astype(jnp.bfloat16)

    sub = 4
    grid = (batch // sub,)
    kfn = functools.partial(_vq_kernel, num_blocks=batch // sub,
                            n_total=n_total, sub=sub)
    out, loss, ps, pc = pl.pallas_call(
        kfn,
        grid=grid,
        in_specs=[
            pl.BlockSpec((sub, emb, hw), lambda b: (b, 0, 0)),
            pl.BlockSpec((NUM_SHAPE_CODES, HALF_DIM + 1), lambda b: (0, 0)),
            pl.BlockSpec((NUM_SHAPE_CODES, HALF_DIM), lambda b: (0, 0)),
            pl.BlockSpec((NUM_COLOR_CODES, HALF_DIM), lambda b: (0, 0)),
        ],
        out_specs=[
            pl.BlockSpec((sub, emb, hw), lambda b: (b, 0, 0)),
            pl.BlockSpec((1, 1), lambda b: (0, 0)),
            pl.BlockSpec((1, 1), lambda b: (0, 0)),
            pl.BlockSpec((1, 1), lambda b: (0, 0)),
        ],
        out_shape=[
            jax.ShapeDtypeStruct((batch, emb, hw), jnp.float32),
            jax.ShapeDtypeStruct((1, 1), jnp.float32),
            jax.ShapeDtypeStruct((1, 1), jnp.float32),
            jax.ShapeDtypeStruct((1, 1), jnp.float32),
        ],
        scratch_shapes=[
            pltpu.VMEM((NUM_SHAPE_CODES, HALF_DIM), jnp.float32),
            pltpu.VMEM((NUM_COLOR_CODES, HALF_DIM), jnp.float32),
            pltpu.VMEM((1, hw), jnp.float32),
        ],
        compiler_params=pltpu.CompilerParams(
            dimension_semantics=("arbitrary",),
        ),
    )(x3, ws_aug, W_shape, W_color)

    quantized = out.reshape(batch, emb, h, w)
    return (quantized, loss[0, 0], ps[0, 0], pc[0, 0])


# speedup vs baseline: 1.3266x; 1.0030x over previous
import functools

import jax
import jax.numpy as jnp
from jax.experimental import pallas as pl
from jax.experimental.pallas import tpu as pltpu

NUM_SHAPE_CODES = 1024
NUM_COLOR_CODES = 16
EMBEDDING_DIM = 256
HALF_DIM = 128
COMMITMENT_COST = 0.25


def _vq_kernel(x_ref, wsa_ref, ws_ref, wc_ref,
               out_ref, loss_ref, ps_ref, pc_ref,
               cs_acc, cc_acc, ls_acc, *, num_blocks, n_total, sub):
    b = pl.program_id(0)
    wsa = wsa_ref[...]
    ws = ws_ref[...]
    wc = wc_ref[...]

    @pl.when(b == 0)
    def _init():
        cs_acc[...] = jnp.zeros_like(cs_acc)
        cc_acc[...] = jnp.zeros_like(cc_acc)
        ls_acc[...] = jnp.zeros_like(ls_acc)

    cs_part = cs_acc[...]
    cc_part = cc_acc[...]
    ls_part = ls_acc[...]
    for t in range(sub):
        x = x_ref[t]
        xs = x[:HALF_DIM, :]
        xc = x[HALF_DIM:, :]
        p = x.shape[1]

        xs_aug = jnp.concatenate(
            [xs.astype(jnp.bfloat16), jnp.ones((1, p), jnp.bfloat16)], axis=0)
        ds = jax.lax.dot_general(wsa, xs_aug, (((1,), (0,)), ((), ())),
                                 preferred_element_type=jnp.float32)
        ms = jnp.min(ds, axis=0, keepdims=True)
        es = jnp.where(ds == ms, 1.0, 0.0)

        w2c = jnp.sum(wc * wc, axis=1, keepdims=True)
        x2c = jnp.sum(xc * xc, axis=0, keepdims=True)
        sc = jax.lax.dot_general(wc, xc, (((1,), (0,)), ((), ())),
                                 preferred_element_type=jnp.float32)
        dc = (x2c + w2c) - 2.0 * sc
        mc = jnp.min(dc, axis=0, keepdims=True)
        iota_c = jax.lax.broadcasted_iota(jnp.int32, (NUM_COLOR_CODES, p), 0)
        idx_c = jnp.min(jnp.where(dc == mc, iota_c, NUM_COLOR_CODES),
                        axis=0, keepdims=True)
        ec = jnp.where(iota_c == idx_c, 1.0, 0.0)

        qs = jax.lax.dot_general(ws, es, (((0,), (0,)), ((), ())),
                                 preferred_element_type=jnp.float32)
        qc = jax.lax.dot_general(wc, ec, (((0,), (0,)), ((), ())),
                                 preferred_element_type=jnp.float32)
        out_ref[t, :HALF_DIM, :] = qs
        out_ref[t, HALF_DIM:, :] = qc

        x2s_row = jnp.sum(xs * xs, axis=0, keepdims=True)
        ls_part = ls_part + (ms + x2s_row + mc)

        es_chunks = [es[:, k * HALF_DIM:(k + 1) * HALF_DIM]
                     for k in range(p // HALF_DIM)]
        ec_chunks = [ec[:, k * HALF_DIM:(k + 1) * HALF_DIM]
                     for k in range(p // HALF_DIM)]
        while len(es_chunks) > 1:
            es_chunks = [a + b for a, b in zip(es_chunks[::2], es_chunks[1::2])]
            ec_chunks = [a + b for a, b in zip(ec_chunks[::2], ec_chunks[1::2])]
        cs_part = cs_part + es_chunks[0]
        cc_part = cc_part + ec_chunks[0]
    cs_acc[...] = cs_part
    cc_acc[...] = cc_part
    ls_acc[...] = ls_part

    @pl.when(b == num_blocks - 1)
    def _finalize():
        probs_s = jnp.sum(cs_acc[...], axis=1, keepdims=True) * (1.0 / n_total)
        probs_c = jnp.sum(cc_acc[...], axis=1, keepdims=True) * (1.0 / n_total)
        ps_ref[...] = jnp.exp(-jnp.sum(probs_s * jnp.log(probs_s + 1e-10))).reshape(1, 1)
        pc_ref[...] = jnp.exp(-jnp.sum(probs_c * jnp.log(probs_c + 1e-10))).reshape(1, 1)
        scale = (1.0 + COMMITMENT_COST) / (n_total * EMBEDDING_DIM)
        loss_ref[...] = (jnp.sum(ls_acc[...]) * scale).reshape(1, 1)


def kernel(inputs, W_shape, W_color):
    batch, emb, h, w = inputs.shape
    hw = h * w
    n_total = batch * hw
    x3 = inputs.reshape(batch, emb, hw)
    ws_aug = jnp.concatenate(
        [W_shape * -2.0, jnp.sum(W_shape * W_shape, axis=1, keepdims=True)],
        axis=1).astype(jnp.bfloat16)

    sub = 4
    grid = (batch // sub,)
    kfn = functools.partial(_vq_kernel, num_blocks=batch // sub,
                            n_total=n_total, sub=sub)
    out, loss, ps, pc = pl.pallas_call(
        kfn,
        grid=grid,
        in_specs=[
            pl.BlockSpec((sub, emb, hw), lambda b: (b, 0, 0)),
            pl.BlockSpec((NUM_SHAPE_CODES, HALF_DIM + 1), lambda b: (0, 0)),
            pl.BlockSpec((NUM_SHAPE_CODES, HALF_DIM), lambda b: (0, 0)),
            pl.BlockSpec((NUM_COLOR_CODES, HALF_DIM), lambda b: (0, 0)),
        ],
        out_specs=[
            pl.BlockSpec((sub, emb, hw), lambda b: (b, 0, 0)),
            pl.BlockSpec((1, 1), lambda b: (0, 0)),
            pl.BlockSpec((1, 1), lambda b: (0, 0)),
            pl.BlockSpec((1, 1), lambda b: (0, 0)),
        ],
        out_shape=[
            jax.ShapeDtypeStruct((batch, emb, hw), jnp.float32),
            jax.ShapeDtypeStruct((1, 1), jnp.float32),
            jax.ShapeDtypeStruct((1, 1), jnp.float32),
            jax.ShapeDtypeStruct((1, 1), jnp.float32),
        ],
        scratch_shapes=[
            pltpu.VMEM((NUM_SHAPE_CODES, HALF_DIM), jnp.float32),
            pltpu.VMEM((NUM_COLOR_CODES, HALF_DIM), jnp.float32),
            pltpu.VMEM((1, hw), jnp.float32),
        ],
        compiler_params=pltpu.CompilerParams(
            dimension_semantics=("arbitrary",),
        ),
    )(x3, ws_aug, W_shape, W_color)

    quantized = out.reshape(batch, emb, h, w)
    return (quantized, loss[0, 0], ps[0, 0], pc[0, 0])
